# baseline (speedup 1.0000x reference)
import functools

import jax
import jax.numpy as jnp
from jax import lax
from jax.experimental import pallas as pl
from jax.experimental.pallas import tpu as pltpu
from jax.experimental.pallas import tpu_sc as plsc

N_CHAR = 1000
HIDDEN = 64
BATCH = 4096
SEQ = 200
LANES = 16


def kernel(inputs, table, s_cls_token):
    info = plsc.get_sparse_core_info()
    nc, ns = info.num_cores, info.num_subcores
    nw = nc * ns
    bt = BATCH // nw

    idx_flat = inputs.reshape(BATCH * SEQ).astype(jnp.int32)
    table_flat = table.reshape(N_CHAR * HIDDEN)
    cls_plane = jnp.broadcast_to(
        s_cls_token.reshape(HIDDEN // 8, 1, 8, 1).astype(jnp.float32),
        (HIDDEN // 8, 1, 8, bt))

    mesh = plsc.VectorSubcoreMesh(core_axis_name="c", subcore_axis_name="s")

    @functools.partial(
        pl.kernel,
        mesh=mesh,
        out_type=jax.ShapeDtypeStruct((SEQ + 1, HIDDEN // 8, nw, 8, bt), jnp.float32),
        scratch_types=[
            pltpu.VMEM((N_CHAR * HIDDEN,), jnp.float32),
            pltpu.VMEM((bt * SEQ,), jnp.int32),
            pltpu.VMEM((HIDDEN // 8, 1, 8, bt), jnp.float32),
            pltpu.VMEM((HIDDEN // 8, 1, 8, bt), jnp.float32),
            pltpu.VMEM((HIDDEN // 8, 1, 8, bt), jnp.float32),
            pltpu.SemaphoreType.DMA((3,)),
        ],
        compiler_params=pltpu.CompilerParams(use_tc_tiling_on_sc=False, needs_layout_passes=False),
    )
    def emb_kernel(idx_hbm, table_hbm, cls_hbm, out_hbm,
                   table_v, idx_v, stage_a, stage_b, stage_c, wsem):
        wid = lax.axis_index("s") * nc + lax.axis_index("c")
        base = wid * bt

        pltpu.sync_copy(table_hbm, table_v)
        pltpu.sync_copy(cls_hbm, stage_c)
        pltpu.sync_copy(idx_hbm.at[pl.ds(base * SEQ, bt * SEQ)], idx_v)

        lane = lax.iota(jnp.int32, LANES)
        lane_row = lane * SEQ

        def compute_plane(sm1, stg):
            for cg in range(bt // LANES):
                idx16 = plsc.load_gather(
                    idx_v, [lane_row + (cg * LANES * SEQ + sm1)])
                pidx = idx16 * HIDDEN
                for h in range(HIDDEN):
                    v = plsc.load_gather(table_v, [pidx + h])
                    stg[h // 8, 0, h % 8, pl.ds(cg * LANES, LANES)] = v

        def dst(s):
            return out_hbm.at[s, pl.ds(0, HIDDEN // 8), pl.ds(wid, 1)]

        def start_write(s, stg, k):
            pltpu.async_copy(stg, dst(s), wsem.at[k])

        def wait_write(s, stg, k):
            pltpu.make_async_copy(stg, dst(s), wsem.at[k]).wait()

        start_write(0, stage_c, 2)

        compute_plane(0, stage_a)
        start_write(1, stage_a, 0)
        compute_plane(1, stage_b)
        start_write(2, stage_b, 1)

        @pl.loop(1, SEQ // 2)
        def pair(g):
            s_a = 2 * g + 1
            wait_write(s_a - 2, stage_a, 0)
            compute_plane(s_a - 1, stage_a)
            start_write(s_a, stage_a, 0)
            wait_write(s_a - 1, stage_b, 1)
            compute_plane(s_a, stage_b)
            start_write(s_a + 1, stage_b, 1)

        wait_write(SEQ - 1, stage_a, 0)
        wait_write(SEQ, stage_b, 1)
        wait_write(0, stage_c, 2)

    out5d = emb_kernel(idx_flat, table_flat, cls_plane)
    return out5d.transpose(2, 4, 0, 1, 3).reshape(BATCH, SEQ + 1, HIDDEN)


# speedup vs baseline: 1.8291x; 1.8291x over previous
import functools

import jax
import jax.numpy as jnp
from jax import lax
from jax.experimental import pallas as pl
from jax.experimental.pallas import tpu as pltpu
from jax.experimental.pallas import tpu_sc as plsc

N_CHAR = 1000
HIDDEN = 64
BATCH = 4096
SEQ = 200
LANES = 16


def kernel(inputs, table, s_cls_token):
    info = plsc.get_sparse_core_info()
    nc, ns = info.num_cores, info.num_subcores
    nw = nc * ns
    bt = BATCH // nw

    idx_flat = inputs.reshape(BATCH * SEQ).astype(jnp.int32)
    table_flat = table.reshape(N_CHAR * HIDDEN)
    cls_plane = jnp.broadcast_to(
        s_cls_token.reshape(HIDDEN // 8, 1, 8, 1).astype(jnp.float32),
        (HIDDEN // 8, 1, 8, bt))

    mesh = plsc.VectorSubcoreMesh(core_axis_name="c", subcore_axis_name="s")

    @functools.partial(
        pl.kernel,
        mesh=mesh,
        out_type=jax.ShapeDtypeStruct((SEQ + 1, HIDDEN // 8, nw, 8, bt), jnp.float32),
        scratch_types=[
            pltpu.VMEM((N_CHAR * HIDDEN,), jnp.float32),
            pltpu.VMEM((bt * SEQ,), jnp.int32),
            pltpu.VMEM((HIDDEN // 8, 1, 8, bt), jnp.float32),
            pltpu.VMEM((HIDDEN // 8, 1, 8, bt), jnp.float32),
            pltpu.VMEM((HIDDEN // 8, 1, 8, bt), jnp.float32),
            pltpu.SemaphoreType.DMA((3,)),
        ],
        compiler_params=pltpu.CompilerParams(use_tc_tiling_on_sc=False, needs_layout_passes=False),
    )
    def emb_kernel(idx_hbm, table_hbm, cls_hbm, out_hbm,
                   table_v, idx_v, stage_a, stage_b, stage_c, wsem):
        wid = lax.axis_index("s") * nc + lax.axis_index("c")
        base = wid * bt

        pltpu.sync_copy(table_hbm, table_v)
        pltpu.sync_copy(cls_hbm, stage_c)
        pltpu.sync_copy(idx_hbm.at[pl.ds(base * SEQ, bt * SEQ)], idx_v)

        lane = lax.iota(jnp.int32, LANES)
        lane_row = lane * SEQ

        LAG = 8

        def compute_plane(sm1, stg):
            def store(cg, h, v):
                stg[h // 8, 0, h % 8, pl.ds(cg * LANES, LANES)] = v

            for cg in range(bt // LANES):
                idx16 = plsc.load_gather(
                    idx_v, [lane_row + (cg * LANES * SEQ + sm1)])
                pidx = idx16 * HIDDEN
                vals = [None] * HIDDEN
                for h in range(HIDDEN):
                    vals[h] = plsc.load_gather(table_v, [pidx + h])
                    if h >= LAG:
                        store(cg, h - LAG, vals[h - LAG])
                for h in range(HIDDEN - LAG, HIDDEN):
                    store(cg, h, vals[h])

        def dst(s):
            return out_hbm.at[s, pl.ds(0, HIDDEN // 8), pl.ds(wid, 1)]

        def start_write(s, stg, k):
            pltpu.async_copy(stg, dst(s), wsem.at[k])

        def wait_write(s, stg, k):
            pltpu.make_async_copy(stg, dst(s), wsem.at[k]).wait()

        start_write(0, stage_c, 2)

        compute_plane(0, stage_a)
        start_write(1, stage_a, 0)
        compute_plane(1, stage_b)
        start_write(2, stage_b, 1)

        @pl.loop(1, SEQ // 2)
        def pair(g):
            s_a = 2 * g + 1
            wait_write(s_a - 2, stage_a, 0)
            compute_plane(s_a - 1, stage_a)
            start_write(s_a, stage_a, 0)
            wait_write(s_a - 1, stage_b, 1)
            compute_plane(s_a, stage_b)
            start_write(s_a + 1, stage_b, 1)

        wait_write(SEQ - 1, stage_a, 0)
        wait_write(SEQ, stage_b, 1)
        wait_write(0, stage_c, 2)

    out5d = emb_kernel(idx_flat, table_flat, cls_plane)
    return out5d.transpose(2, 4, 0, 1, 3).reshape(BATCH, SEQ + 1, HIDDEN)


# speedup vs baseline: 8.4000x; 4.5926x over previous
import functools

import jax
import jax.numpy as jnp
from jax import lax
from jax.experimental import pallas as pl
from jax.experimental.pallas import tpu as pltpu
from jax.experimental.pallas import tpu_sc as plsc

N_CHAR = 1000
HIDDEN = 64
BATCH = 4096
SEQ = 200
LANES = 16


def kernel(inputs, table, s_cls_token):
    info = plsc.get_sparse_core_info()
    nc, ns = info.num_cores, info.num_subcores
    nw = nc * ns
    bt = BATCH // nw

    idx_flat = inputs.reshape(BATCH * SEQ).astype(jnp.int32)
    table_t_flat = table.T.reshape(HIDDEN * N_CHAR)
    cls_plane = jnp.broadcast_to(
        s_cls_token.reshape(HIDDEN // 8, 1, 8, 1).astype(jnp.float32),
        (HIDDEN // 8, 1, 8, bt))

    mesh = plsc.VectorSubcoreMesh(core_axis_name="c", subcore_axis_name="s")

    @functools.partial(
        pl.kernel,
        mesh=mesh,
        out_type=jax.ShapeDtypeStruct((SEQ + 1, HIDDEN // 8, nw, 8, bt), jnp.float32),
        scratch_types=[
            pltpu.VMEM((N_CHAR * HIDDEN,), jnp.float32),
            pltpu.VMEM((bt * SEQ,), jnp.int32),
            pltpu.VMEM((HIDDEN // 8, 1, 8, bt), jnp.float32),
            pltpu.VMEM((HIDDEN // 8, 1, 8, bt), jnp.float32),
            pltpu.VMEM((HIDDEN // 8, 1, 8, bt), jnp.float32),
            pltpu.SemaphoreType.DMA((3,)),
        ],
        compiler_params=pltpu.CompilerParams(use_tc_tiling_on_sc=False, needs_layout_passes=False),
    )
    def emb_kernel(idx_hbm, table_hbm, cls_hbm, out_hbm,
                   table_v, idx_v, stage_a, stage_b, stage_c, wsem):
        wid = lax.axis_index("s") * nc + lax.axis_index("c")
        base = wid * bt

        pltpu.sync_copy(table_hbm, table_v)
        pltpu.sync_copy(cls_hbm, stage_c)
        pltpu.sync_copy(idx_hbm.at[pl.ds(base * SEQ, bt * SEQ)], idx_v)

        lane = lax.iota(jnp.int32, LANES)
        lane_row = lane * SEQ

        LAG = 8

        def compute_plane(sm1, stg):
            def store(cg, h, v):
                stg[h // 8, 0, h % 8, pl.ds(cg * LANES, LANES)] = v

            for cg in range(bt // LANES):
                idx16 = plsc.load_gather(
                    idx_v, [lane_row + (cg * LANES * SEQ + sm1)])
                vals = [None] * HIDDEN
                for h in range(HIDDEN):
                    vals[h] = plsc.load_gather(table_v, [idx16 + h * N_CHAR])
                    if h >= LAG:
                        store(cg, h - LAG, vals[h - LAG])
                for h in range(HIDDEN - LAG, HIDDEN):
                    store(cg, h, vals[h])

        def dst(s):
            return out_hbm.at[s, pl.ds(0, HIDDEN // 8), pl.ds(wid, 1)]

        def start_write(s, stg, k):
            pltpu.async_copy(stg, dst(s), wsem.at[k])

        def wait_write(s, stg, k):
            pltpu.make_async_copy(stg, dst(s), wsem.at[k]).wait()

        start_write(0, stage_c, 2)

        compute_plane(0, stage_a)
        start_write(1, stage_a, 0)
        compute_plane(1, stage_b)
        start_write(2, stage_b, 1)

        @pl.loop(1, SEQ // 2)
        def pair(g):
            s_a = 2 * g + 1
            wait_write(s_a - 2, stage_a, 0)
            compute_plane(s_a - 1, stage_a)
            start_write(s_a, stage_a, 0)
            wait_write(s_a - 1, stage_b, 1)
            compute_plane(s_a, stage_b)
            start_write(s_a + 1, stage_b, 1)

        wait_write(SEQ - 1, stage_a, 0)
        wait_write(SEQ, stage_b, 1)
        wait_write(0, stage_c, 2)

    out5d = emb_kernel(idx_flat, table_t_flat, cls_plane)
    return out5d.transpose(2, 4, 0, 1, 3).reshape(BATCH, SEQ + 1, HIDDEN)


# speedup vs baseline: 9.2416x; 1.1002x over previous
import functools

import jax
import jax.numpy as jnp
from jax import lax
from jax.experimental import pallas as pl
from jax.experimental.pallas import tpu as pltpu
from jax.experimental.pallas import tpu_sc as plsc

N_CHAR = 1000
HIDDEN = 64
BATCH = 4096
SEQ = 200
LANES = 16


def kernel(inputs, table, s_cls_token):
    info = plsc.get_sparse_core_info()
    nc, ns = info.num_cores, info.num_subcores
    nw = nc * ns
    bt = BATCH // nw

    idx4 = (inputs.astype(jnp.int32).T
            .reshape(SEQ // 8, 8, BATCH // 128, 128)
            .transpose(0, 2, 1, 3))
    table_t_flat = table.T.reshape(HIDDEN * N_CHAR)
    cls_plane = jnp.broadcast_to(
        s_cls_token.reshape(HIDDEN // 8, 1, 8, 1).astype(jnp.float32),
        (HIDDEN // 8, 1, 8, bt))

    mesh = plsc.VectorSubcoreMesh(core_axis_name="c", subcore_axis_name="s")

    @functools.partial(
        pl.kernel,
        mesh=mesh,
        out_type=jax.ShapeDtypeStruct((SEQ + 1, HIDDEN // 8, nw, 8, bt), jnp.float32),
        scratch_types=[
            pltpu.VMEM((N_CHAR * HIDDEN,), jnp.float32),
            pltpu.VMEM((SEQ // 8, 1, 8, bt), jnp.int32),
            pltpu.VMEM((HIDDEN // 8, 1, 8, bt), jnp.float32),
            pltpu.VMEM((HIDDEN // 8, 1, 8, bt), jnp.float32),
            pltpu.VMEM((HIDDEN // 8, 1, 8, bt), jnp.float32),
            pltpu.SemaphoreType.DMA((3,)),
        ],
        compiler_params=pltpu.CompilerParams(use_tc_tiling_on_sc=False, needs_layout_passes=False),
    )
    def emb_kernel(idx_hbm, table_hbm, cls_hbm, out_hbm,
                   table_v, idx_v, stage_a, stage_b, stage_c, wsem):
        wid = lax.axis_index("s") * nc + lax.axis_index("c")
        base = wid * bt

        pltpu.sync_copy(table_hbm, table_v)
        pltpu.sync_copy(cls_hbm, stage_c)
        pltpu.sync_copy(idx_hbm.at[pl.ds(0, SEQ // 8), pl.ds(wid, 1)], idx_v)

        LAG = 8

        def compute_plane(sm1, stg):
            ts = sm1 // 8
            sr = sm1 % 8

            def store(cg, h, v):
                stg[h // 8, 0, h % 8, pl.ds(cg * LANES, LANES)] = v

            for cg in range(bt // LANES):
                idx16 = idx_v[ts, 0, sr, pl.ds(cg * LANES, LANES)]
                vals = [None] * HIDDEN
                for h in range(HIDDEN):
                    vals[h] = plsc.load_gather(table_v, [idx16 + h * N_CHAR])
                    if h >= LAG:
                        store(cg, h - LAG, vals[h - LAG])
                for h in range(HIDDEN - LAG, HIDDEN):
                    store(cg, h, vals[h])

        def dst(s):
            return out_hbm.at[s, pl.ds(0, HIDDEN // 8), pl.ds(wid, 1)]

        def start_write(s, stg, k):
            pltpu.async_copy(stg, dst(s), wsem.at[k])

        def wait_write(s, stg, k):
            pltpu.make_async_copy(stg, dst(s), wsem.at[k]).wait()

        start_write(0, stage_c, 2)

        compute_plane(0, stage_a)
        start_write(1, stage_a, 0)
        compute_plane(1, stage_b)
        start_write(2, stage_b, 1)

        @pl.loop(1, SEQ // 2)
        def pair(g):
            s_a = 2 * g + 1
            wait_write(s_a - 2, stage_a, 0)
            compute_plane(s_a - 1, stage_a)
            start_write(s_a, stage_a, 0)
            wait_write(s_a - 1, stage_b, 1)
            compute_plane(s_a, stage_b)
            start_write(s_a + 1, stage_b, 1)

        wait_write(SEQ - 1, stage_a, 0)
        wait_write(SEQ, stage_b, 1)
        wait_write(0, stage_c, 2)

    out5d = emb_kernel(idx4, table_t_flat, cls_plane)
    return out5d.transpose(2, 4, 0, 1, 3).reshape(BATCH, SEQ + 1, HIDDEN)
